# baseline (device time: 78769 ns/iter reference)
import jax
import jax.numpy as jnp
from jax import lax
from jax.experimental import pallas as pl
from jax.experimental.pallas import tpu as pltpu

N_DEV = 32
ENABLE_COMM = True
B, SQ, SKV = 2, 512, 512
HQ_PER, DH = 8, 64
DM = 768
HB = HQ_PER * DH
ROWS = B * SQ
CH = ROWS // N_DEV


def kernel(x, Wq, K_ext, V_ext, Wo):
    i = lax.axis_index("i")
    Wq_i = lax.dynamic_slice(Wq, (0, i * HB), (DM, HB))
    Wo_i = lax.dynamic_slice(Wo, (i * HB, 0), (HB, DM))

    def body(x_ref, wq_ref, k_ref, v_ref, wo_ref, out_ref,
             ctx_ref, p_ref, red_ref, rs_buf, ag_buf,
             send1, recv1, send2, recv2):
        me = lax.axis_index("i")

        x2d = x_ref[:].reshape(ROWS, DM).astype(jnp.bfloat16)
        q2d = jnp.dot(x2d, wq_ref[:].astype(jnp.bfloat16),
                      preferred_element_type=jnp.float32)
        qi = lax.broadcasted_iota(jnp.int32, (SQ, SKV), 0)
        ki = lax.broadcasted_iota(jnp.int32, (SQ, SKV), 1)
        mask = (jnp.abs(qi - ki) <= 128) | (ki < 32) | (qi < 32)
        neg = jnp.where(mask, 0.0, -1e9).astype(jnp.float32)
        for b in range(B):
            for h in range(HQ_PER):
                q = q2d[b * SQ:(b + 1) * SQ, h * DH:(h + 1) * DH]
                k = k_ref[b, :, h, :].astype(jnp.bfloat16)
                v = v_ref[b, :, h, :].astype(jnp.bfloat16)
                s = lax.dot_general(q.astype(jnp.bfloat16), k,
                                    (((1,), (1,)), ((), ())),
                                    preferred_element_type=jnp.float32)
                s = s * 0.125 + neg
                m = jnp.max(s, axis=1, keepdims=True)
                w = jnp.exp(s - m)
                w = w / jnp.sum(w, axis=1, keepdims=True)
                ctx_ref[b, :, h * DH:(h + 1) * DH] = jnp.dot(
                    w.astype(jnp.bfloat16), v,
                    preferred_element_type=jnp.float32).astype(jnp.bfloat16)
        p = jnp.dot(ctx_ref[:].reshape(ROWS, HB), wo_ref[:].astype(jnp.bfloat16),
                    preferred_element_type=jnp.float32)
        p_ref[:] = p.astype(jnp.bfloat16).reshape(N_DEV, CH, DM)

        if not ENABLE_COMM:
            out_ref[:] = p
            return

        sends1 = []
        for off in range(1, N_DEV):
            tgt = lax.rem(me + off, N_DEV)
            r = pltpu.make_async_remote_copy(
                src_ref=p_ref.at[tgt],
                dst_ref=rs_buf.at[me],
                send_sem=send1.at[off],
                recv_sem=recv1.at[me],
                device_id=(tgt,),
                device_id_type=pl.DeviceIdType.MESH,
            )
            r.start()
            sends1.append(r)
        rs_buf[me] = p_ref[me]

        for off in range(1, N_DEV):
            src = lax.rem(me + off, N_DEV)
            rw = pltpu.make_async_remote_copy(
                src_ref=p_ref.at[0],
                dst_ref=rs_buf.at[src],
                send_sem=send1.at[0],
                recv_sem=recv1.at[src],
                device_id=(me,),
                device_id_type=pl.DeviceIdType.MESH,
            )
            rw.wait_recv()
        red_ref[:] = jnp.sum(rs_buf[:].astype(jnp.float32),
                             axis=0).astype(jnp.bfloat16)

        sends2 = []
        for off in range(1, N_DEV):
            tgt = lax.rem(me + off, N_DEV)
            r = pltpu.make_async_remote_copy(
                src_ref=red_ref,
                dst_ref=ag_buf.at[me],
                send_sem=send2.at[off],
                recv_sem=recv2.at[me],
                device_id=(tgt,),
                device_id_type=pl.DeviceIdType.MESH,
            )
            r.start()
            sends2.append(r)
        ag_buf[me] = red_ref[:]

        for off in range(1, N_DEV):
            src = lax.rem(me + off, N_DEV)
            rw = pltpu.make_async_remote_copy(
                src_ref=red_ref,
                dst_ref=ag_buf.at[src],
                send_sem=send2.at[0],
                recv_sem=recv2.at[src],
                device_id=(me,),
                device_id_type=pl.DeviceIdType.MESH,
            )
            rw.wait_recv()
        out_ref[:] = ag_buf[:].reshape(ROWS, DM).astype(jnp.float32)

        for r in sends1:
            r.wait_send()
        for r in sends2:
            r.wait_send()

    out = pl.pallas_call(
        body,
        out_shape=jax.ShapeDtypeStruct((ROWS, DM), jnp.float32),
        in_specs=[pl.BlockSpec(memory_space=pltpu.VMEM)] * 5,
        out_specs=pl.BlockSpec(memory_space=pltpu.VMEM),
        scratch_shapes=[
            pltpu.VMEM((B, SQ, HB), jnp.bfloat16),
            pltpu.VMEM((N_DEV, CH, DM), jnp.bfloat16),
            pltpu.VMEM((CH, DM), jnp.bfloat16),
            pltpu.VMEM((N_DEV, CH, DM), jnp.bfloat16),
            pltpu.VMEM((N_DEV, CH, DM), jnp.bfloat16),
            pltpu.SemaphoreType.DMA((N_DEV,)),
            pltpu.SemaphoreType.DMA((N_DEV,)),
            pltpu.SemaphoreType.DMA((N_DEV,)),
            pltpu.SemaphoreType.DMA((N_DEV,)),
        ],
    )(x, Wq_i, K_ext, V_ext, Wo_i)
    return out.reshape(B, SQ, DM)


# device time: 77159 ns/iter; 1.0209x vs baseline; 1.0209x over previous
import jax
import jax.numpy as jnp
from jax import lax
from jax.experimental import pallas as pl
from jax.experimental.pallas import tpu as pltpu

N_DEV = 32
ENABLE_COMM = True
B, SQ, SKV = 2, 512, 512
HQ_PER, DH = 8, 64
DM = 768
HB = HQ_PER * DH
ROWS = B * SQ
CH = ROWS // N_DEV
NSUB = 2
CHS = CH // NSUB


def kernel(x, Wq, K_ext, V_ext, Wo):
    i = lax.axis_index("i")
    Wq_i = lax.dynamic_slice(Wq, (0, i * HB), (DM, HB))
    Wo_i = lax.dynamic_slice(Wo, (i * HB, 0), (HB, DM))

    def body(x_ref, wq_ref, k_ref, v_ref, wo_ref, out_ref,
             ctx_ref, p_ref, red_ref, rs_buf, ag_buf,
             send1, recv1, send2, recv2):
        me = lax.axis_index("i")

        x2d = x_ref[:].reshape(ROWS, DM).astype(jnp.bfloat16)
        q2d = jnp.dot(x2d, wq_ref[:].astype(jnp.bfloat16),
                      preferred_element_type=jnp.float32)
        qi = lax.broadcasted_iota(jnp.int32, (SQ, SKV), 0)
        ki = lax.broadcasted_iota(jnp.int32, (SQ, SKV), 1)
        mask = (jnp.abs(qi - ki) <= 128) | (ki < 32) | (qi < 32)
        neg = jnp.where(mask, 0.0, -1e9).astype(jnp.float32)
        for b in range(B):
            for h in range(HQ_PER):
                q = q2d[b * SQ:(b + 1) * SQ, h * DH:(h + 1) * DH]
                k = k_ref[b, :, h, :].astype(jnp.bfloat16)
                v = v_ref[b, :, h, :].astype(jnp.bfloat16)
                s = lax.dot_general(q.astype(jnp.bfloat16), k,
                                    (((1,), (1,)), ((), ())),
                                    preferred_element_type=jnp.float32)
                s = s * 0.125 + neg
                m = jnp.max(s, axis=1, keepdims=True)
                w = jnp.exp(s - m)
                w = w / jnp.sum(w, axis=1, keepdims=True)
                ctx_ref[b, :, h * DH:(h + 1) * DH] = jnp.dot(
                    w.astype(jnp.bfloat16), v,
                    preferred_element_type=jnp.float32).astype(jnp.bfloat16)
        p = jnp.dot(ctx_ref[:].reshape(ROWS, HB), wo_ref[:].astype(jnp.bfloat16),
                    preferred_element_type=jnp.float32)
        p_ref[:] = p.astype(jnp.bfloat16).reshape(N_DEV, NSUB, CHS, DM)

        if not ENABLE_COMM:
            out_ref[:] = p
            return

        sends1 = []
        for sub in range(NSUB):
            for off in range(1, N_DEV):
                tgt = lax.rem(me + off, N_DEV)
                r = pltpu.make_async_remote_copy(
                    src_ref=p_ref.at[tgt, sub],
                    dst_ref=rs_buf.at[me, sub],
                    send_sem=send1.at[sub, off],
                    recv_sem=recv1.at[sub, me],
                    device_id=(tgt,),
                    device_id_type=pl.DeviceIdType.MESH,
                )
                r.start()
                sends1.append(r)
        rs_buf[me] = p_ref[me]

        sends2 = []
        for sub in range(NSUB):
            for off in range(1, N_DEV):
                src = lax.rem(me + off, N_DEV)
                rw = pltpu.make_async_remote_copy(
                    src_ref=p_ref.at[0, 0],
                    dst_ref=rs_buf.at[src, sub],
                    send_sem=send1.at[0, 0],
                    recv_sem=recv1.at[sub, src],
                    device_id=(me,),
                    device_id_type=pl.DeviceIdType.MESH,
                )
                rw.wait_recv()
            red_ref[sub] = jnp.sum(rs_buf[:, sub].astype(jnp.float32),
                                   axis=0).astype(jnp.bfloat16)
            for off in range(1, N_DEV):
                tgt = lax.rem(me + off, N_DEV)
                r = pltpu.make_async_remote_copy(
                    src_ref=red_ref.at[sub],
                    dst_ref=ag_buf.at[me, sub],
                    send_sem=send2.at[sub, off],
                    recv_sem=recv2.at[sub, me],
                    device_id=(tgt,),
                    device_id_type=pl.DeviceIdType.MESH,
                )
                r.start()
                sends2.append(r)
            ag_buf[me, sub] = red_ref[sub]

        for sub in range(NSUB):
            for off in range(1, N_DEV):
                src = lax.rem(me + off, N_DEV)
                rw = pltpu.make_async_remote_copy(
                    src_ref=red_ref.at[0],
                    dst_ref=ag_buf.at[src, sub],
                    send_sem=send2.at[0, 0],
                    recv_sem=recv2.at[sub, src],
                    device_id=(me,),
                    device_id_type=pl.DeviceIdType.MESH,
                )
                rw.wait_recv()
        out_ref[:] = ag_buf[:].reshape(ROWS, DM).astype(jnp.float32)

        for r in sends1:
            r.wait_send()
        for r in sends2:
            r.wait_send()

    out = pl.pallas_call(
        body,
        out_shape=jax.ShapeDtypeStruct((ROWS, DM), jnp.float32),
        in_specs=[pl.BlockSpec(memory_space=pltpu.VMEM)] * 5,
        out_specs=pl.BlockSpec(memory_space=pltpu.VMEM),
        scratch_shapes=[
            pltpu.VMEM((B, SQ, HB), jnp.bfloat16),
            pltpu.VMEM((N_DEV, NSUB, CHS, DM), jnp.bfloat16),
            pltpu.VMEM((NSUB, CHS, DM), jnp.bfloat16),
            pltpu.VMEM((N_DEV, NSUB, CHS, DM), jnp.bfloat16),
            pltpu.VMEM((N_DEV, NSUB, CHS, DM), jnp.bfloat16),
            pltpu.SemaphoreType.DMA((NSUB, N_DEV)),
            pltpu.SemaphoreType.DMA((NSUB, N_DEV)),
            pltpu.SemaphoreType.DMA((NSUB, N_DEV)),
            pltpu.SemaphoreType.DMA((NSUB, N_DEV)),
        ],
    )(x, Wq_i, K_ext, V_ext, Wo_i)
    return out.reshape(B, SQ, DM)


# device time: 66612 ns/iter; 1.1825x vs baseline; 1.1583x over previous
import jax
import jax.numpy as jnp
from jax import lax
from jax.experimental import pallas as pl
from jax.experimental.pallas import tpu as pltpu

N_DEV = 32
ENABLE_COMM = True
SKIP_COMPUTE = False
B, SQ, SKV = 2, 512, 512
HQ_PER, DH = 8, 64
DM = 768
HB = HQ_PER * DH
ROWS = B * SQ
NSUB = 4
SPB = NSUB // B
CHS = ROWS // (N_DEV * NSUB)


def kernel(x, Wq, K_ext, V_ext, Wo):
    i = lax.axis_index("i")
    Wq_i = lax.dynamic_slice(Wq, (0, i * HB), (DM, HB))
    Wo_i = lax.dynamic_slice(Wo, (i * HB, 0), (HB, DM))

    def body(x_ref, wq_ref, k_ref, v_ref, wo_ref, out_ref,
             ctx_ref, p_ref, red_ref, rs_buf, ag_buf,
             send1, recv1, send2, recv2):
        me = lax.axis_index("i")

        barrier_sem = pltpu.get_barrier_semaphore()
        for off in range(1, N_DEV):
            tgt = lax.rem(me + off, N_DEV)
            pl.semaphore_signal(barrier_sem, inc=1, device_id=(tgt,),
                                device_id_type=pl.DeviceIdType.MESH)

        def attn_batch(b):
            xb = x_ref[b].astype(jnp.bfloat16)
            qb = jnp.dot(xb, wq_ref[:].astype(jnp.bfloat16),
                         preferred_element_type=jnp.float32)
            qi = lax.broadcasted_iota(jnp.int32, (SQ, SKV), 0)
            ki = lax.broadcasted_iota(jnp.int32, (SQ, SKV), 1)
            mask = (jnp.abs(qi - ki) <= 128) | (ki < 32) | (qi < 32)
            neg = jnp.where(mask, 0.0, -1e9).astype(jnp.float32)
            for h in range(HQ_PER):
                q = qb[:, h * DH:(h + 1) * DH]
                k = k_ref[b, :, h, :].astype(jnp.bfloat16)
                v = v_ref[b, :, h, :].astype(jnp.bfloat16)
                s = lax.dot_general(q.astype(jnp.bfloat16), k,
                                    (((1,), (1,)), ((), ())),
                                    preferred_element_type=jnp.float32)
                s = s * 0.125 + neg
                m = jnp.max(s, axis=1, keepdims=True)
                w = jnp.exp(s - m)
                w = w / jnp.sum(w, axis=1, keepdims=True)
                ctx_ref[:, h * DH:(h + 1) * DH] = jnp.dot(
                    w.astype(jnp.bfloat16), v,
                    preferred_element_type=jnp.float32).astype(jnp.bfloat16)
            pb = jnp.dot(ctx_ref[:], wo_ref[:].astype(jnp.bfloat16),
                         preferred_element_type=jnp.float32)
            pb = pb.astype(jnp.bfloat16)
            for qtr in range(SPB):
                p_ref[:, b * SPB + qtr] = pb[
                    qtr * (SQ // SPB):(qtr + 1) * (SQ // SPB)
                ].reshape(N_DEV, CHS, DM)

        sends1 = []

        def rs_send(sub):
            for off in range(1, N_DEV):
                tgt = lax.rem(me + off, N_DEV)
                r = pltpu.make_async_remote_copy(
                    src_ref=p_ref.at[tgt, sub],
                    dst_ref=rs_buf.at[me, sub],
                    send_sem=send1.at[sub, off],
                    recv_sem=recv1.at[sub, me],
                    device_id=(tgt,),
                    device_id_type=pl.DeviceIdType.MESH,
                )
                r.start()
                sends1.append(r)
            rs_buf[me, sub] = p_ref[me, sub]

        if SKIP_COMPUTE:
            p_ref[:] = x_ref[:].reshape(N_DEV, NSUB, CHS, DM).astype(jnp.bfloat16)
            pl.semaphore_wait(barrier_sem, N_DEV - 1)
            for sub in range(NSUB):
                rs_send(sub)
        else:
            attn_batch(0)
            pl.semaphore_wait(barrier_sem, N_DEV - 1)
            for qtr in range(SPB):
                rs_send(qtr)
            attn_batch(1)
            for qtr in range(SPB):
                rs_send(SPB + qtr)

        if not ENABLE_COMM:
            out_ref[:] = p_ref[:].reshape(ROWS, DM).astype(jnp.float32)
            return

        sends2 = []
        for sub in range(NSUB):
            for off in range(1, N_DEV):
                src = lax.rem(me + off, N_DEV)
                rw = pltpu.make_async_remote_copy(
                    src_ref=p_ref.at[0, 0],
                    dst_ref=rs_buf.at[src, sub],
                    send_sem=send1.at[0, 0],
                    recv_sem=recv1.at[sub, src],
                    device_id=(me,),
                    device_id_type=pl.DeviceIdType.MESH,
                )
                rw.wait_recv()
            red_ref[sub] = jnp.sum(rs_buf[:, sub].astype(jnp.float32),
                                   axis=0).astype(jnp.bfloat16)
            for off in range(1, N_DEV):
                tgt = lax.rem(me + off, N_DEV)
                r = pltpu.make_async_remote_copy(
                    src_ref=red_ref.at[sub],
                    dst_ref=ag_buf.at[me, sub],
                    send_sem=send2.at[sub, off],
                    recv_sem=recv2.at[sub, me],
                    device_id=(tgt,),
                    device_id_type=pl.DeviceIdType.MESH,
                )
                r.start()
                sends2.append(r)
            ag_buf[me, sub] = red_ref[sub]

        for sub in range(NSUB):
            for off in range(1, N_DEV):
                src = lax.rem(me + off, N_DEV)
                rw = pltpu.make_async_remote_copy(
                    src_ref=red_ref.at[0],
                    dst_ref=ag_buf.at[src, sub],
                    send_sem=send2.at[0, 0],
                    recv_sem=recv2.at[sub, src],
                    device_id=(me,),
                    device_id_type=pl.DeviceIdType.MESH,
                )
                rw.wait_recv()
            out_ref[pl.ds(sub * (ROWS // NSUB), ROWS // NSUB), :] = (
                ag_buf[:, sub].reshape(ROWS // NSUB, DM).astype(jnp.float32))

        for r in sends1:
            r.wait_send()
        for r in sends2:
            r.wait_send()

    out = pl.pallas_call(
        body,
        out_shape=jax.ShapeDtypeStruct((ROWS, DM), jnp.float32),
        in_specs=[pl.BlockSpec(memory_space=pltpu.VMEM)] * 5,
        out_specs=pl.BlockSpec(memory_space=pltpu.VMEM),
        scratch_shapes=[
            pltpu.VMEM((SQ, HB), jnp.bfloat16),
            pltpu.VMEM((N_DEV, NSUB, CHS, DM), jnp.bfloat16),
            pltpu.VMEM((NSUB, CHS, DM), jnp.bfloat16),
            pltpu.VMEM((N_DEV, NSUB, CHS, DM), jnp.bfloat16),
            pltpu.VMEM((N_DEV, NSUB, CHS, DM), jnp.bfloat16),
            pltpu.SemaphoreType.DMA((NSUB, N_DEV)),
            pltpu.SemaphoreType.DMA((NSUB, N_DEV)),
            pltpu.SemaphoreType.DMA((NSUB, N_DEV)),
            pltpu.SemaphoreType.DMA((NSUB, N_DEV)),
        ],
        compiler_params=pltpu.CompilerParams(collective_id=0),
    )(x, Wq_i, K_ext, V_ext, Wo_i)
    return out.reshape(B, SQ, DM)
